# baseline (speedup 1.0000x reference)
import functools

import jax
import jax.numpy as jnp
from jax import lax
from jax.experimental import pallas as pl
from jax.experimental.pallas import tpu as pltpu
from jax.experimental.pallas import tpu_sc as plsc

N = 10000
E = 320000
H = 128
C = 10
EPS = 1e-5

NC = 2
NS = 16
NW = NC * NS
EPT = E // NW
EB = 80
NEB = EPT // EB
RZ = 640
RZ_LAST = N - 15 * RZ

_mesh = plsc.VectorSubcoreMesh(
    core_axis_name="c", subcore_axis_name="s", num_cores=NC, num_subcores=NS)
_sc_params = pltpu.CompilerParams(needs_layout_passes=False)

f32 = jnp.float32
i32 = jnp.int32


ZR1 = 80
ZR2 = 40


def _fill_zeros_1d(zbuf):
    def fz(i, c):
        zbuf[pl.ds(i * 16, 16)] = jnp.zeros((16,), f32)
        return c

    lax.fori_loop(0, ZR1 // 16, fz, 0)


def _fill_zeros_2d(zbuf):
    def fz(i, c):
        for k in range(H // 16):
            zbuf[i, pl.ds(k * 16, 16)] = jnp.zeros((16,), f32)
        return c

    lax.fori_loop(0, ZR2, fz, 0)


def _chunks(s, zr, fn):
    @pl.when(s < 15)
    def _():
        for j in range(RZ // zr):
            fn(s * RZ + j * zr)

    @pl.when(s == 15)
    def _():
        for j in range(RZ_LAST // zr):
            fn(15 * RZ + j * zr)


def _zero_rows_1d(zbuf, acc, s):
    _chunks(s, ZR1,
            lambda off: pltpu.sync_copy(zbuf, acc.at[pl.ds(off, ZR1)]))


def _zero_rows_2d(zbuf, acc, s):
    _chunks(s, ZR2,
            lambda off: pltpu.sync_copy(zbuf, acc.at[pl.ds(off, ZR2)]))


def _writeback_scalar(acc, out_flat, base, s, bounce):
    def fn(off):
        pltpu.sync_copy(acc.at[pl.ds(off, ZR1)], bounce)
        pltpu.sync_copy(bounce, out_flat.at[pl.ds(base + off, ZR1)])

    _chunks(s, ZR1, fn)


def _writeback_rows(acc, out, c, s, bounce):
    def fn(off):
        pltpu.sync_copy(acc.at[pl.ds(off, ZR2)], bounce)
        pltpu.sync_copy(bounce, out.at[c, pl.ds(off, ZR2)])

    _chunks(s, ZR2, fn)


def _counts_body(row_hbm, out_hbm, rowv, onesv, zbuf, acc, sem):
    c = lax.axis_index("c")
    s = lax.axis_index("s")
    wid = c * NS + s
    _fill_zeros_1d(zbuf)
    _zero_rows_1d(zbuf, acc, s)
    for k in range(EB // 16):
        onesv[pl.ds(k * 16, 16)] = jnp.ones((16,), f32)
    plsc.subcore_barrier()
    ebase = wid * EPT

    def blk(j, carry):
        off = ebase + j * EB
        pltpu.sync_copy(row_hbm.at[pl.ds(off, EB)], rowv)
        pltpu.sync_copy(onesv, acc.at[rowv], add=True)
        return carry

    lax.fori_loop(0, NEB, blk, 0)
    plsc.subcore_barrier()
    _writeback_scalar(acc, out_hbm, c * N, s, zbuf)


_counts_call = pl.kernel(
    _counts_body,
    out_type=jax.ShapeDtypeStruct((NC * N,), f32),
    mesh=_mesh,
    compiler_params=_sc_params,
    scratch_types=[
        pltpu.VMEM((EB,), i32),
        pltpu.VMEM((EB,), f32),
        pltpu.VMEM((ZR1,), f32),
        pltpu.VMEM_SHARED((N,), f32),
        pltpu.SemaphoreType.DMA,
    ],
)


EBB = 96
NBLK = EPT // EBB
ETAIL = EPT - NBLK * EBB
NBUF = 3
IR = 4


def _scale_rows(rows_blk, ewv_blk, nedge):
    def scale(eg, cc):
        ew16 = ewv_blk[pl.ds(eg * 16, 16)]
        blk16 = rows_blk.at[pl.ds(eg * 16, 16)]
        for l in range(16):
            wv = jnp.take(ew16, jnp.full((16,), l, i32))
            for k in range(H // 16):
                sl = pl.ds(k * 16, 16)
                blk16[l, sl] = blk16[l, sl] * wv
        return cc

    lax.fori_loop(0, nedge // 16, scale, 0)


def _make_segsum_body(weighted):
    def body(*refs):
        if weighted:
            (g_hbm, row_hbm, col_hbm, ew_hbm, out_hbm,
             rowv, colv, ewv, rows_v, colt, rowt, ewt, rowst,
             zbuf, acc, isem, gsem, ssem) = refs
        else:
            (g_hbm, row_hbm, col_hbm, out_hbm,
             rowv, colv, rows_v, colt, rowt, rowst,
             zbuf, acc, isem, gsem, ssem) = refs
        c = lax.axis_index("c")
        s = lax.axis_index("s")
        wid = c * NS + s
        _fill_zeros_2d(zbuf)
        _zero_rows_2d(zbuf, acc, s)
        plsc.subcore_barrier()
        ebase = wid * EPT

        def idx_copies(j, b):
            off = ebase + j * EBB
            yield (col_hbm.at[pl.ds(off, EBB)], colv.at[b])
            yield (row_hbm.at[pl.ds(off, EBB)], rowv.at[b])
            if weighted:
                yield (ew_hbm.at[pl.ds(off, EBB)], ewv.at[b])

        def istart(j):
            b = lax.rem(j, IR)
            for src, dst in idx_copies(j, b):
                pltpu.async_copy(src, dst, isem.at[b])

        def iwait(j):
            b = lax.rem(j, IR)
            for src, dst in idx_copies(j, b):
                pltpu.make_async_copy(src, dst, isem.at[b]).wait()

        def gstart(j):
            b = lax.rem(j, NBUF)
            pltpu.async_copy(g_hbm.at[colv.at[lax.rem(j, IR)]],
                             rows_v.at[b], gsem.at[b])

        def gwait(j):
            b = lax.rem(j, NBUF)
            pltpu.make_async_copy(g_hbm.at[colv.at[lax.rem(j, IR)]],
                                  rows_v.at[b], gsem.at[b]).wait()

        def sstart(j):
            b = lax.rem(j, NBUF)
            if weighted:
                _scale_rows(rows_v.at[b], ewv.at[lax.rem(j, IR)], EBB)
            pltpu.async_copy(rows_v.at[b], acc.at[rowv.at[lax.rem(j, IR)]],
                             ssem.at[b], add=True)

        def sdrain(j):
            b = lax.rem(j, NBUF)
            pltpu.make_async_copy(rows_v.at[b],
                                  acc.at[rowv.at[lax.rem(j, IR)]],
                                  ssem.at[b]).wait()

        istart(0)
        istart(1)
        iwait(0)
        gstart(0)

        def blk(i, carry):
            @pl.when(i >= 2)
            def _():
                sdrain(i - 2)

            @pl.when(i + 2 < NBLK)
            def _():
                istart(i + 2)

            @pl.when(i + 1 < NBLK)
            def _():
                iwait(i + 1)
                gstart(i + 1)

            gwait(i)
            sstart(i)
            return carry

        lax.fori_loop(0, NBLK, blk, 0)
        sdrain(NBLK - 2)
        sdrain(NBLK - 1)

        if ETAIL:
            off = ebase + NBLK * EBB
            pltpu.sync_copy(col_hbm.at[pl.ds(off, ETAIL)], colt)
            pltpu.async_copy(g_hbm.at[colt], rowst, gsem.at[0]).wait()
            pltpu.sync_copy(row_hbm.at[pl.ds(off, ETAIL)], rowt)
            if weighted:
                pltpu.sync_copy(ew_hbm.at[pl.ds(off, ETAIL)], ewt)
                _scale_rows(rowst, ewt, ETAIL)
            pltpu.sync_copy(rowst, acc.at[rowt], add=True)

        plsc.subcore_barrier()
        _writeback_rows(acc, out_hbm, c, s, zbuf)

    return body


def _make_segsum_call(weighted):
    scratch = [
        pltpu.VMEM((IR, EBB), i32),
        pltpu.VMEM((IR, EBB), i32),
    ]
    if weighted:
        scratch.append(pltpu.VMEM((IR, EBB), f32))
    scratch += [
        pltpu.VMEM((NBUF, EBB, H), f32),
        pltpu.VMEM((ETAIL,), i32),
        pltpu.VMEM((ETAIL,), i32),
    ]
    if weighted:
        scratch.append(pltpu.VMEM((ETAIL,), f32))
    scratch += [
        pltpu.VMEM((ETAIL, H), f32),
        pltpu.VMEM((ZR2, H), f32),
        pltpu.VMEM_SHARED((N, H), f32),
        pltpu.SemaphoreType.DMA((IR,)),
        pltpu.SemaphoreType.DMA((NBUF,)),
        pltpu.SemaphoreType.DMA((NBUF,)),
    ]
    return pl.kernel(
        _make_segsum_body(weighted),
        out_type=jax.ShapeDtypeStruct((NC, N, H), f32),
        mesh=_mesh,
        compiler_params=_sc_params,
        scratch_types=scratch,
    )


_segsum_u = _make_segsum_call(False)
_segsum_w = _make_segsum_call(True)


def _edgeatt_body(a_hbm, b_hbm, row_hbm, col_hbm,
                  ewc_hbm, ewo_hbm, deg_hbm,
                  av, bv, rowf, colf, ewcf, ewof, rowv, zbuf,
                  degc, dego, sem):
    c = lax.axis_index("c")
    s = lax.axis_index("s")
    wid = c * NS + s
    _fill_zeros_1d(zbuf)
    _zero_rows_1d(zbuf, degc, s)
    _zero_rows_1d(zbuf, dego, s)
    pltpu.sync_copy(a_hbm, av)
    pltpu.sync_copy(b_hbm, bv)
    ebase = wid * EPT
    pltpu.sync_copy(row_hbm.at[pl.ds(ebase, EPT)], rowf)
    pltpu.sync_copy(col_hbm.at[pl.ds(ebase, EPT)], colf)
    plsc.subcore_barrier()

    def att(i, carry):
        sl = pl.ds(i * 16, 16)
        r2 = rowf[sl] * 2
        c2 = colf[sl] * 2
        l0 = plsc.load_gather(av, [r2]) + plsc.load_gather(bv, [c2])
        l1 = plsc.load_gather(av, [r2 + 1]) + plsc.load_gather(bv, [c2 + 1])
        m = jnp.maximum(l0, l1)
        e0 = jnp.exp(l0 - m)
        e1 = jnp.exp(l1 - m)
        inv = 1.0 / (e0 + e1)
        ewcf[sl] = e0 * inv
        ewof[sl] = e1 * inv
        return carry

    lax.fori_loop(0, EPT // 16, att, 0)

    def degblk(j, carry):
        for k in range(EB // 16):
            rowv[pl.ds(k * 16, 16)] = rowf[pl.ds(j * EB + k * 16, 16)]
        pltpu.sync_copy(ewcf.at[pl.ds(j * EB, EB)], degc.at[rowv], add=True)
        pltpu.sync_copy(ewof.at[pl.ds(j * EB, EB)], dego.at[rowv], add=True)
        return carry

    lax.fori_loop(0, NEB, degblk, 0)

    pltpu.sync_copy(ewcf, ewc_hbm.at[pl.ds(ebase, EPT)])
    pltpu.sync_copy(ewof, ewo_hbm.at[pl.ds(ebase, EPT)])
    plsc.subcore_barrier()
    _writeback_scalar(degc, deg_hbm, (c * 2 + 0) * N, s, zbuf)
    _writeback_scalar(dego, deg_hbm, (c * 2 + 1) * N, s, zbuf)


_edgeatt_call = pl.kernel(
    _edgeatt_body,
    out_type=(
        jax.ShapeDtypeStruct((E,), f32),
        jax.ShapeDtypeStruct((E,), f32),
        jax.ShapeDtypeStruct((NC * 2 * N,), f32),
    ),
    mesh=_mesh,
    compiler_params=_sc_params,
    scratch_types=[
        pltpu.VMEM((2 * N,), f32),
        pltpu.VMEM((2 * N,), f32),
        pltpu.VMEM((EPT,), i32),
        pltpu.VMEM((EPT,), i32),
        pltpu.VMEM((EPT,), f32),
        pltpu.VMEM((EPT,), f32),
        pltpu.VMEM((EB,), i32),
        pltpu.VMEM((ZR1,), f32),
        pltpu.VMEM_SHARED((N,), f32),
        pltpu.VMEM_SHARED((N,), f32),
        pltpu.SemaphoreType.DMA,
    ],
)


_PB = 80
_PROWS = 320


def _permgather_body(src_hbm, perm_hbm, out_hbm, idxv, rows_v, sem):
    c = lax.axis_index("c")
    s = lax.axis_index("s")
    wid = c * NS + s
    base = wid * _PROWS

    def blk(j, carry):
        off = base + j * _PB
        pltpu.sync_copy(perm_hbm.at[pl.ds(off, _PB)], idxv)
        pltpu.async_copy(src_hbm.at[idxv], rows_v, sem).wait()
        pltpu.sync_copy(rows_v, out_hbm.at[pl.ds(off, _PB)])
        return carry

    nb = jnp.where(wid == NW - 1, (N - (NW - 1) * _PROWS) // _PB,
                   _PROWS // _PB)
    lax.fori_loop(0, nb, blk, 0)


_permgather_call = pl.kernel(
    _permgather_body,
    out_type=jax.ShapeDtypeStruct((N, H), f32),
    mesh=_mesh,
    compiler_params=_sc_params,
    scratch_types=[
        pltpu.VMEM((_PB,), i32),
        pltpu.VMEM((_PB, H), f32),
        pltpu.SemaphoreType.DMA,
    ],
)


def _bn(x, g, b):
    mu = jnp.mean(x, axis=0, keepdims=True)
    var = jnp.mean((x - mu) ** 2, axis=0, keepdims=True)
    return (x - mu) * lax.rsqrt(var + EPS) * g + b


def _mm(a, w):
    return jnp.dot(a, w, preferred_element_type=f32)


def _tc1_body(x_ref, bfg, bfb, Wf, b0g, b0b, W0, cnt_ref, g0_ref, dinv_ref):
    x = x_ref[...]
    xn = _bn(x, bfg[...], bfb[...])
    x1 = jnp.maximum(_mm(xn, Wf[...]), 0.0)
    deg = cnt_ref[..., 0:1] + cnt_ref[..., 1:2] + 1.0
    dinv = lax.rsqrt(deg)
    dinv_ref[...] = dinv
    h = _mm(_bn(x1, b0g[...], b0b[...]), W0[...])
    g0_ref[...] = dinv * h


def _tc1(x, bfg, bfb, Wf, b0g, b0b, W0, cnt_t):
    return pl.pallas_call(
        _tc1_body,
        out_shape=(
            jax.ShapeDtypeStruct((N, H), f32),
            jax.ShapeDtypeStruct((N, 1), f32),
        ),
    )(x, bfg, bfb, Wf, b0g, b0b, W0, cnt_t)


def _tcmid_body(s_ref, g_ref, dinv_ref, bprev, bng, bnb, W, gout_ref):
    dinv = dinv_ref[...]
    out = dinv * (s_ref[0] + s_ref[1] + g_ref[...]) + bprev[...]
    xk = jnp.maximum(out, 0.0)
    gout_ref[...] = dinv * _mm(_bn(xk, bng[...], bnb[...]), W[...])


def _tcmid(s, g, dinv, bprev, bng, bnb, W):
    return pl.pallas_call(
        _tcmid_body,
        out_shape=jax.ShapeDtypeStruct((N, H), f32),
    )(s, g, dinv, bprev, bng, bnb, W)


def _tc4_body(s_ref, g_ref, dinv_ref, bprev, Wea_t, Wea_b, eab, Wna, nab,
              bncg, bncb, ctxW, bnog, bnob, objW,
              a_ref, b_ref, hc_ref, ho_ref):
    dinv = dinv_ref[...]
    x4 = jnp.maximum(dinv * (s_ref[0] + s_ref[1] + g_ref[...]) + bprev[...],
                     0.0)
    a_ref[...] = _mm(x4, Wea_t[...]) + eab[...]
    b_ref[...] = _mm(x4, Wea_b[...])
    na = _mm(x4, Wna[...]) + nab[...]
    na = na - jnp.max(na, axis=-1, keepdims=True)
    na = jnp.exp(na)
    na = na / jnp.sum(na, axis=-1, keepdims=True)
    xc = na[:, 0:1] * x4
    xo = na[:, 1:2] * x4
    hc_ref[...] = _mm(_bn(xc, bncg[...], bncb[...]), ctxW[...])
    ho_ref[...] = _mm(_bn(xo, bnog[...], bnob[...]), objW[...])


def _tc4(s, g, dinv, bprev, Wea_t, Wea_b, eab, Wna, nab,
         bncg, bncb, ctxW, bnog, bnob, objW):
    return pl.pallas_call(
        _tc4_body,
        out_shape=(
            jax.ShapeDtypeStruct((N, 2), f32),
            jax.ShapeDtypeStruct((N, 2), f32),
            jax.ShapeDtypeStruct((N, H), f32),
            jax.ShapeDtypeStruct((N, H), f32),
        ),
    )(s, g, dinv, bprev, Wea_t, Wea_b, eab, Wna, nab,
      bncg, bncb, ctxW, bnog, bnob, objW)


def _tc5_body(degc_ref, dego_ref, hc_ref, ho_ref,
              gc_ref, go_ref, dinvc_ref, dinvo_ref):
    dc = degc_ref[..., 0:1] + degc_ref[..., 1:2] + 1.0
    do = dego_ref[..., 0:1] + dego_ref[..., 1:2] + 1.0
    dinvc = lax.rsqrt(dc)
    dinvo = lax.rsqrt(do)
    dinvc_ref[...] = dinvc
    dinvo_ref[...] = dinvo
    gc_ref[...] = dinvc * hc_ref[...]
    go_ref[...] = dinvo * ho_ref[...]


def _tc5(degc_t, dego_t, hc, ho):
    return pl.pallas_call(
        _tc5_body,
        out_shape=(
            jax.ShapeDtypeStruct((N, H), f32),
            jax.ShapeDtypeStruct((N, H), f32),
            jax.ShapeDtypeStruct((N, 1), f32),
            jax.ShapeDtypeStruct((N, 1), f32),
        ),
    )(degc_t, dego_t, hc, ho)


def _head(x, g1, b1, W1, bb1, g2, b2, W2, bb2):
    x = _bn(x, g1, b1)
    x = jnp.maximum(_mm(x, W1) + bb1, 0.0)
    x = _bn(x, g2, b2)
    lg = _mm(x, W2) + bb2
    sh = lg - jnp.max(lg, axis=-1, keepdims=True)
    return sh - jnp.log(jnp.sum(jnp.exp(sh), axis=-1, keepdims=True))


def _tc6_body(s_ref, g_ref, dinv_ref, bconv,
              g1, b1, W1, bb1, g2, b2, W2, bb2,
              x_ref, logis_ref):
    x = jnp.maximum(dinv_ref[...] * (s_ref[0] + s_ref[1] + g_ref[...])
                    + bconv[...], 0.0)
    x_ref[...] = x
    logis_ref[...] = _head(x, g1[...], b1[...], W1[...], bb1[...],
                           g2[...], b2[...], W2[...], bb2[...])


def _tc6(s, g, dinv, bconv, g1, b1, W1, bb1, g2, b2, W2, bb2):
    return pl.pallas_call(
        _tc6_body,
        out_shape=(
            jax.ShapeDtypeStruct((N, H), f32),
            jax.ShapeDtypeStruct((N, C), f32),
        ),
    )(s, g, dinv, bconv, g1, b1, W1, bb1, g2, b2, W2, bb2)


def _tc7_body(xcp_ref, xo_ref, g1, b1, W1, bb1, g2, b2, W2, bb2, logis_ref):
    xco = xcp_ref[...] + xo_ref[...]
    logis_ref[...] = _head(xco, g1[...], b1[...], W1[...], bb1[...],
                           g2[...], b2[...], W2[...], bb2[...])


def _tc7(xcp, xo, g1, b1, W1, bb1, g2, b2, W2, bb2):
    return pl.pallas_call(
        _tc7_body,
        out_shape=jax.ShapeDtypeStruct((N, C), f32),
    )(xcp, xo, g1, b1, W1, bb1, g2, b2, W2, bb2)


def kernel(x, edge_index, params):
    p = params
    row = edge_index[0]
    col = edge_index[1]

    cnt = _counts_call(row).reshape(NC, N)
    cnt_t = jnp.transpose(cnt)

    g0, dinv = _tc1(x, p['bn_feat_g'], p['bn_feat_b'], p['conv_feat_W'],
                    p['bn0_g'], p['bn0_b'], p['conv0_W'], cnt_t)

    s0 = _segsum_u(g0, row, col)
    g1 = _tcmid(s0, g0, dinv, p['conv0_b'], p['bn1_g'], p['bn1_b'],
                p['conv1_W'])
    s1 = _segsum_u(g1, row, col)
    g2 = _tcmid(s1, g1, dinv, p['conv1_b'], p['bn2_g'], p['bn2_b'],
                p['conv2_W'])
    s2 = _segsum_u(g2, row, col)

    Wea = p['edge_att_W']
    a_att, b_att, hc, ho = _tc4(
        s2, g2, dinv, p['conv2_b'], Wea[:H], Wea[H:], p['edge_att_b'],
        p['node_att_W'], p['node_att_b'],
        p['bnc_g'], p['bnc_b'], p['ctx_W'],
        p['bno_g'], p['bno_b'], p['obj_W'])

    ewc, ewo, deg = _edgeatt_call(a_att.reshape(-1), b_att.reshape(-1),
                                  row, col)
    deg = deg.reshape(NC, 2, N)
    degc_t = jnp.transpose(deg[:, 0])
    dego_t = jnp.transpose(deg[:, 1])

    gc, go, dinvc, dinvo = _tc5(degc_t, dego_t, hc, ho)

    sc = _segsum_w(gc, row, col, ewc)
    so = _segsum_w(go, row, col, ewo)

    xc, xc_logis = _tc6(sc, gc, dinvc, p['ctx_b'],
                        p['c_bn1_g'], p['c_bn1_b'], p['c_fc1_W'], p['c_fc1_b'],
                        p['c_bn2_g'], p['c_bn2_b'], p['c_fc2_W'], p['c_fc2_b'])
    xo, xo_logis = _tc6(so, go, dinvo, p['obj_b'],
                        p['o_bn1_g'], p['o_bn1_b'], p['o_fc1_W'], p['o_fc1_b'],
                        p['o_bn2_g'], p['o_bn2_b'], p['o_fc2_W'], p['o_fc2_b'])

    perm = jax.random.permutation(jax.random.key(42), N).astype(i32)
    xcp = _permgather_call(xc, perm)

    xco_logis = _tc7(xcp, xo,
                     p['co_bn1_g'], p['co_bn1_b'], p['co_fc1_W'],
                     p['co_fc1_b'], p['co_bn2_g'], p['co_bn2_b'],
                     p['co_fc2_W'], p['co_fc2_b'])

    return (xc_logis, xo_logis, xco_logis)


# speedup vs baseline: 1.3313x; 1.3313x over previous
import functools

import jax
import jax.numpy as jnp
from jax import lax
from jax.experimental import pallas as pl
from jax.experimental.pallas import tpu as pltpu
from jax.experimental.pallas import tpu_sc as plsc

N = 10000
E = 320000
H = 128
C = 10
EPS = 1e-5

NC = 2
NS = 16
NW = NC * NS
EPT = E // NW
EB = 80
NEB = EPT // EB
RZ = 640
RZ_LAST = N - 15 * RZ

_mesh = plsc.VectorSubcoreMesh(
    core_axis_name="c", subcore_axis_name="s", num_cores=NC, num_subcores=NS)
_sc_params = pltpu.CompilerParams(needs_layout_passes=False)

f32 = jnp.float32
i32 = jnp.int32


ZR1 = 80
ZR2 = 40


def _fill_zeros_1d(zbuf):
    def fz(i, c):
        zbuf[pl.ds(i * 16, 16)] = jnp.zeros((16,), f32)
        return c

    lax.fori_loop(0, ZR1 // 16, fz, 0)


def _fill_zeros_2d(zbuf):
    def fz(i, c):
        for k in range(H // 16):
            zbuf[i, pl.ds(k * 16, 16)] = jnp.zeros((16,), f32)
        return c

    lax.fori_loop(0, ZR2, fz, 0)


def _chunks(s, zr, fn):
    @pl.when(s < 15)
    def _():
        for j in range(RZ // zr):
            fn(s * RZ + j * zr)

    @pl.when(s == 15)
    def _():
        for j in range(RZ_LAST // zr):
            fn(15 * RZ + j * zr)


def _zero_rows_1d(zbuf, acc, s):
    _chunks(s, ZR1,
            lambda off: pltpu.sync_copy(zbuf, acc.at[pl.ds(off, ZR1)]))


def _zero_rows_2d(zbuf, acc, s):
    _chunks(s, ZR2,
            lambda off: pltpu.sync_copy(zbuf, acc.at[pl.ds(off, ZR2)]))


def _writeback_scalar(acc, out_flat, base, s, bounce):
    def fn(off):
        pltpu.sync_copy(acc.at[pl.ds(off, ZR1)], bounce)
        pltpu.sync_copy(bounce, out_flat.at[pl.ds(base + off, ZR1)])

    _chunks(s, ZR1, fn)


def _writeback_rows(acc, out, c, s, bounce):
    def fn(off):
        pltpu.sync_copy(acc.at[pl.ds(off, ZR2)], bounce)
        pltpu.sync_copy(bounce, out.at[c, pl.ds(off, ZR2)])

    _chunks(s, ZR2, fn)


def _counts_body(row_hbm, out_hbm, rowv, onesv, zbuf, acc, sem):
    c = lax.axis_index("c")
    s = lax.axis_index("s")
    wid = c * NS + s
    _fill_zeros_1d(zbuf)
    _zero_rows_1d(zbuf, acc, s)
    for k in range(EB // 16):
        onesv[pl.ds(k * 16, 16)] = jnp.ones((16,), f32)
    plsc.subcore_barrier()
    ebase = wid * EPT

    def blk(j, carry):
        off = ebase + j * EB
        pltpu.sync_copy(row_hbm.at[pl.ds(off, EB)], rowv)
        pltpu.sync_copy(onesv, acc.at[rowv], add=True)
        return carry

    lax.fori_loop(0, NEB, blk, 0)
    plsc.subcore_barrier()
    _writeback_scalar(acc, out_hbm, c * N, s, zbuf)


_counts_call = pl.kernel(
    _counts_body,
    out_type=jax.ShapeDtypeStruct((NC * N,), f32),
    mesh=_mesh,
    compiler_params=_sc_params,
    scratch_types=[
        pltpu.VMEM((EB,), i32),
        pltpu.VMEM((EB,), f32),
        pltpu.VMEM((ZR1,), f32),
        pltpu.VMEM_SHARED((N,), f32),
        pltpu.SemaphoreType.DMA,
    ],
)


EBB = 96
NBLK = EPT // EBB
ETAIL = EPT - NBLK * EBB
NBUF = 3
IR = 4


def _scale_rows(rows_blk, ewv_blk, nedge):
    @plsc.parallel_loop(0, nedge // 16, unroll=2)
    def scale(eg):
        ew16 = ewv_blk[pl.ds(eg * 16, 16)]
        blk16 = rows_blk.at[pl.ds(eg * 16, 16)]
        for l in range(16):
            wv = jnp.take(ew16, jnp.full((16,), l, i32))
            for k in range(H // 16):
                sl = pl.ds(k * 16, 16)
                blk16[l, sl] = blk16[l, sl] * wv


def _make_segsum_body(weighted):
    def body(*refs):
        if weighted:
            (g_hbm, row_hbm, col_hbm, ew_hbm, out_hbm,
             rowv, colv, ewv, rows_v, colt, rowt, ewt, rowst,
             zbuf, acc, isem, gsem, ssem) = refs
        else:
            (g_hbm, row_hbm, col_hbm, out_hbm,
             rowv, colv, rows_v, colt, rowt, rowst,
             zbuf, acc, isem, gsem, ssem) = refs
        c = lax.axis_index("c")
        s = lax.axis_index("s")
        wid = c * NS + s
        _fill_zeros_2d(zbuf)
        _zero_rows_2d(zbuf, acc, s)
        plsc.subcore_barrier()
        ebase = wid * EPT

        def idx_copies(j, b):
            off = ebase + j * EBB
            yield (col_hbm.at[pl.ds(off, EBB)], colv.at[b])
            yield (row_hbm.at[pl.ds(off, EBB)], rowv.at[b])
            if weighted:
                yield (ew_hbm.at[pl.ds(off, EBB)], ewv.at[b])

        def istart(j):
            b = lax.rem(j, IR)
            for src, dst in idx_copies(j, b):
                pltpu.async_copy(src, dst, isem.at[b])

        def iwait(j):
            b = lax.rem(j, IR)
            for src, dst in idx_copies(j, b):
                pltpu.make_async_copy(src, dst, isem.at[b]).wait()

        def gstart(j):
            b = lax.rem(j, NBUF)
            pltpu.async_copy(g_hbm.at[colv.at[lax.rem(j, IR)]],
                             rows_v.at[b], gsem.at[b])

        def gwait(j):
            b = lax.rem(j, NBUF)
            pltpu.make_async_copy(g_hbm.at[colv.at[lax.rem(j, IR)]],
                                  rows_v.at[b], gsem.at[b]).wait()

        def sstart(j):
            b = lax.rem(j, NBUF)
            if weighted:
                _scale_rows(rows_v.at[b], ewv.at[lax.rem(j, IR)], EBB)
            pltpu.async_copy(rows_v.at[b], acc.at[rowv.at[lax.rem(j, IR)]],
                             ssem.at[b], add=True)

        def sdrain(j):
            b = lax.rem(j, NBUF)
            pltpu.make_async_copy(rows_v.at[b],
                                  acc.at[rowv.at[lax.rem(j, IR)]],
                                  ssem.at[b]).wait()

        istart(0)
        istart(1)
        iwait(0)
        gstart(0)

        def blk(i, carry):
            @pl.when(i >= 2)
            def _():
                sdrain(i - 2)

            @pl.when(i + 2 < NBLK)
            def _():
                istart(i + 2)

            @pl.when(i + 1 < NBLK)
            def _():
                iwait(i + 1)
                gstart(i + 1)

            gwait(i)
            sstart(i)
            return carry

        lax.fori_loop(0, NBLK, blk, 0)
        sdrain(NBLK - 2)
        sdrain(NBLK - 1)

        if ETAIL:
            off = ebase + NBLK * EBB
            pltpu.sync_copy(col_hbm.at[pl.ds(off, ETAIL)], colt)
            pltpu.async_copy(g_hbm.at[colt], rowst, gsem.at[0]).wait()
            pltpu.sync_copy(row_hbm.at[pl.ds(off, ETAIL)], rowt)
            if weighted:
                pltpu.sync_copy(ew_hbm.at[pl.ds(off, ETAIL)], ewt)
                _scale_rows(rowst, ewt, ETAIL)
            pltpu.sync_copy(rowst, acc.at[rowt], add=True)

        plsc.subcore_barrier()
        _writeback_rows(acc, out_hbm, c, s, zbuf)

    return body


def _make_segsum_call(weighted):
    scratch = [
        pltpu.VMEM((IR, EBB), i32),
        pltpu.VMEM((IR, EBB), i32),
    ]
    if weighted:
        scratch.append(pltpu.VMEM((IR, EBB), f32))
    scratch += [
        pltpu.VMEM((NBUF, EBB, H), f32),
        pltpu.VMEM((ETAIL,), i32),
        pltpu.VMEM((ETAIL,), i32),
    ]
    if weighted:
        scratch.append(pltpu.VMEM((ETAIL,), f32))
    scratch += [
        pltpu.VMEM((ETAIL, H), f32),
        pltpu.VMEM((ZR2, H), f32),
        pltpu.VMEM_SHARED((N, H), f32),
        pltpu.SemaphoreType.DMA((IR,)),
        pltpu.SemaphoreType.DMA((NBUF,)),
        pltpu.SemaphoreType.DMA((NBUF,)),
    ]
    return pl.kernel(
        _make_segsum_body(weighted),
        out_type=jax.ShapeDtypeStruct((NC, N, H), f32),
        mesh=_mesh,
        compiler_params=_sc_params,
        scratch_types=scratch,
    )


_segsum_u = _make_segsum_call(False)
_segsum_w = _make_segsum_call(True)


def _edgeatt_body(a_hbm, b_hbm, row_hbm, col_hbm,
                  ewc_hbm, ewo_hbm, deg_hbm,
                  av, bv, rowf, colf, ewcf, ewof, rowv, zbuf,
                  degc, dego, sem):
    c = lax.axis_index("c")
    s = lax.axis_index("s")
    wid = c * NS + s
    _fill_zeros_1d(zbuf)
    _zero_rows_1d(zbuf, degc, s)
    _zero_rows_1d(zbuf, dego, s)
    pltpu.sync_copy(a_hbm, av)
    pltpu.sync_copy(b_hbm, bv)
    ebase = wid * EPT
    pltpu.sync_copy(row_hbm.at[pl.ds(ebase, EPT)], rowf)
    pltpu.sync_copy(col_hbm.at[pl.ds(ebase, EPT)], colf)
    plsc.subcore_barrier()

    def att(i, carry):
        sl = pl.ds(i * 16, 16)
        r2 = rowf[sl] * 2
        c2 = colf[sl] * 2
        l0 = plsc.load_gather(av, [r2]) + plsc.load_gather(bv, [c2])
        l1 = plsc.load_gather(av, [r2 + 1]) + plsc.load_gather(bv, [c2 + 1])
        m = jnp.maximum(l0, l1)
        e0 = jnp.exp(l0 - m)
        e1 = jnp.exp(l1 - m)
        inv = 1.0 / (e0 + e1)
        ewcf[sl] = e0 * inv
        ewof[sl] = e1 * inv
        return carry

    lax.fori_loop(0, EPT // 16, att, 0)

    def degblk(j, carry):
        for k in range(EB // 16):
            rowv[pl.ds(k * 16, 16)] = rowf[pl.ds(j * EB + k * 16, 16)]
        pltpu.sync_copy(ewcf.at[pl.ds(j * EB, EB)], degc.at[rowv], add=True)
        pltpu.sync_copy(ewof.at[pl.ds(j * EB, EB)], dego.at[rowv], add=True)
        return carry

    lax.fori_loop(0, NEB, degblk, 0)

    pltpu.sync_copy(ewcf, ewc_hbm.at[pl.ds(ebase, EPT)])
    pltpu.sync_copy(ewof, ewo_hbm.at[pl.ds(ebase, EPT)])
    plsc.subcore_barrier()
    _writeback_scalar(degc, deg_hbm, (c * 2 + 0) * N, s, zbuf)
    _writeback_scalar(dego, deg_hbm, (c * 2 + 1) * N, s, zbuf)


_edgeatt_call = pl.kernel(
    _edgeatt_body,
    out_type=(
        jax.ShapeDtypeStruct((E,), f32),
        jax.ShapeDtypeStruct((E,), f32),
        jax.ShapeDtypeStruct((NC * 2 * N,), f32),
    ),
    mesh=_mesh,
    compiler_params=_sc_params,
    scratch_types=[
        pltpu.VMEM((2 * N,), f32),
        pltpu.VMEM((2 * N,), f32),
        pltpu.VMEM((EPT,), i32),
        pltpu.VMEM((EPT,), i32),
        pltpu.VMEM((EPT,), f32),
        pltpu.VMEM((EPT,), f32),
        pltpu.VMEM((EB,), i32),
        pltpu.VMEM((ZR1,), f32),
        pltpu.VMEM_SHARED((N,), f32),
        pltpu.VMEM_SHARED((N,), f32),
        pltpu.SemaphoreType.DMA,
    ],
)


_PB = 80
_PROWS = 320


def _permgather_body(src_hbm, perm_hbm, out_hbm, idxv, rows_v, sem):
    c = lax.axis_index("c")
    s = lax.axis_index("s")
    wid = c * NS + s
    base = wid * _PROWS

    def blk(j, carry):
        off = base + j * _PB
        pltpu.sync_copy(perm_hbm.at[pl.ds(off, _PB)], idxv)
        pltpu.async_copy(src_hbm.at[idxv], rows_v, sem).wait()
        pltpu.sync_copy(rows_v, out_hbm.at[pl.ds(off, _PB)])
        return carry

    nb = jnp.where(wid == NW - 1, (N - (NW - 1) * _PROWS) // _PB,
                   _PROWS // _PB)
    lax.fori_loop(0, nb, blk, 0)


_permgather_call = pl.kernel(
    _permgather_body,
    out_type=jax.ShapeDtypeStruct((N, H), f32),
    mesh=_mesh,
    compiler_params=_sc_params,
    scratch_types=[
        pltpu.VMEM((_PB,), i32),
        pltpu.VMEM((_PB, H), f32),
        pltpu.SemaphoreType.DMA,
    ],
)


def _bn(x, g, b):
    mu = jnp.mean(x, axis=0, keepdims=True)
    var = jnp.mean((x - mu) ** 2, axis=0, keepdims=True)
    return (x - mu) * lax.rsqrt(var + EPS) * g + b


def _mm(a, w):
    return jnp.dot(a, w, preferred_element_type=f32)


def _tc1_body(x_ref, bfg, bfb, Wf, b0g, b0b, W0, cnt_ref, g0_ref, dinv_ref):
    x = x_ref[...]
    xn = _bn(x, bfg[...], bfb[...])
    x1 = jnp.maximum(_mm(xn, Wf[...]), 0.0)
    deg = cnt_ref[..., 0:1] + cnt_ref[..., 1:2] + 1.0
    dinv = lax.rsqrt(deg)
    dinv_ref[...] = dinv
    h = _mm(_bn(x1, b0g[...], b0b[...]), W0[...])
    g0_ref[...] = dinv * h


def _tc1(x, bfg, bfb, Wf, b0g, b0b, W0, cnt_t):
    return pl.pallas_call(
        _tc1_body,
        out_shape=(
            jax.ShapeDtypeStruct((N, H), f32),
            jax.ShapeDtypeStruct((N, 1), f32),
        ),
    )(x, bfg, bfb, Wf, b0g, b0b, W0, cnt_t)


def _tcmid_body(s_ref, g_ref, dinv_ref, bprev, bng, bnb, W, gout_ref):
    dinv = dinv_ref[...]
    out = dinv * (s_ref[0] + s_ref[1] + g_ref[...]) + bprev[...]
    xk = jnp.maximum(out, 0.0)
    gout_ref[...] = dinv * _mm(_bn(xk, bng[...], bnb[...]), W[...])


def _tcmid(s, g, dinv, bprev, bng, bnb, W):
    return pl.pallas_call(
        _tcmid_body,
        out_shape=jax.ShapeDtypeStruct((N, H), f32),
    )(s, g, dinv, bprev, bng, bnb, W)


def _tc4_body(s_ref, g_ref, dinv_ref, bprev, Wea_t, Wea_b, eab, Wna, nab,
              bncg, bncb, ctxW, bnog, bnob, objW,
              a_ref, b_ref, hc_ref, ho_ref):
    dinv = dinv_ref[...]
    x4 = jnp.maximum(dinv * (s_ref[0] + s_ref[1] + g_ref[...]) + bprev[...],
                     0.0)
    a_ref[...] = _mm(x4, Wea_t[...]) + eab[...]
    b_ref[...] = _mm(x4, Wea_b[...])
    na = _mm(x4, Wna[...]) + nab[...]
    na = na - jnp.max(na, axis=-1, keepdims=True)
    na = jnp.exp(na)
    na = na / jnp.sum(na, axis=-1, keepdims=True)
    xc = na[:, 0:1] * x4
    xo = na[:, 1:2] * x4
    hc_ref[...] = _mm(_bn(xc, bncg[...], bncb[...]), ctxW[...])
    ho_ref[...] = _mm(_bn(xo, bnog[...], bnob[...]), objW[...])


def _tc4(s, g, dinv, bprev, Wea_t, Wea_b, eab, Wna, nab,
         bncg, bncb, ctxW, bnog, bnob, objW):
    return pl.pallas_call(
        _tc4_body,
        out_shape=(
            jax.ShapeDtypeStruct((N, 2), f32),
            jax.ShapeDtypeStruct((N, 2), f32),
            jax.ShapeDtypeStruct((N, H), f32),
            jax.ShapeDtypeStruct((N, H), f32),
        ),
    )(s, g, dinv, bprev, Wea_t, Wea_b, eab, Wna, nab,
      bncg, bncb, ctxW, bnog, bnob, objW)


def _tc5_body(degc_ref, dego_ref, hc_ref, ho_ref,
              gc_ref, go_ref, dinvc_ref, dinvo_ref):
    dc = degc_ref[..., 0:1] + degc_ref[..., 1:2] + 1.0
    do = dego_ref[..., 0:1] + dego_ref[..., 1:2] + 1.0
    dinvc = lax.rsqrt(dc)
    dinvo = lax.rsqrt(do)
    dinvc_ref[...] = dinvc
    dinvo_ref[...] = dinvo
    gc_ref[...] = dinvc * hc_ref[...]
    go_ref[...] = dinvo * ho_ref[...]


def _tc5(degc_t, dego_t, hc, ho):
    return pl.pallas_call(
        _tc5_body,
        out_shape=(
            jax.ShapeDtypeStruct((N, H), f32),
            jax.ShapeDtypeStruct((N, H), f32),
            jax.ShapeDtypeStruct((N, 1), f32),
            jax.ShapeDtypeStruct((N, 1), f32),
        ),
    )(degc_t, dego_t, hc, ho)


def _head(x, g1, b1, W1, bb1, g2, b2, W2, bb2):
    x = _bn(x, g1, b1)
    x = jnp.maximum(_mm(x, W1) + bb1, 0.0)
    x = _bn(x, g2, b2)
    lg = _mm(x, W2) + bb2
    sh = lg - jnp.max(lg, axis=-1, keepdims=True)
    return sh - jnp.log(jnp.sum(jnp.exp(sh), axis=-1, keepdims=True))


def _tc6_body(s_ref, g_ref, dinv_ref, bconv,
              g1, b1, W1, bb1, g2, b2, W2, bb2,
              x_ref, logis_ref):
    x = jnp.maximum(dinv_ref[...] * (s_ref[0] + s_ref[1] + g_ref[...])
                    + bconv[...], 0.0)
    x_ref[...] = x
    logis_ref[...] = _head(x, g1[...], b1[...], W1[...], bb1[...],
                           g2[...], b2[...], W2[...], bb2[...])


def _tc6(s, g, dinv, bconv, g1, b1, W1, bb1, g2, b2, W2, bb2):
    return pl.pallas_call(
        _tc6_body,
        out_shape=(
            jax.ShapeDtypeStruct((N, H), f32),
            jax.ShapeDtypeStruct((N, C), f32),
        ),
    )(s, g, dinv, bconv, g1, b1, W1, bb1, g2, b2, W2, bb2)


def _tc7_body(xcp_ref, xo_ref, g1, b1, W1, bb1, g2, b2, W2, bb2, logis_ref):
    xco = xcp_ref[...] + xo_ref[...]
    logis_ref[...] = _head(xco, g1[...], b1[...], W1[...], bb1[...],
                           g2[...], b2[...], W2[...], bb2[...])


def _tc7(xcp, xo, g1, b1, W1, bb1, g2, b2, W2, bb2):
    return pl.pallas_call(
        _tc7_body,
        out_shape=jax.ShapeDtypeStruct((N, C), f32),
    )(xcp, xo, g1, b1, W1, bb1, g2, b2, W2, bb2)


def kernel(x, edge_index, params):
    p = params
    row = edge_index[0]
    col = edge_index[1]

    cnt = _counts_call(row).reshape(NC, N)
    cnt_t = jnp.transpose(cnt)

    g0, dinv = _tc1(x, p['bn_feat_g'], p['bn_feat_b'], p['conv_feat_W'],
                    p['bn0_g'], p['bn0_b'], p['conv0_W'], cnt_t)

    s0 = _segsum_u(g0, row, col)
    g1 = _tcmid(s0, g0, dinv, p['conv0_b'], p['bn1_g'], p['bn1_b'],
                p['conv1_W'])
    s1 = _segsum_u(g1, row, col)
    g2 = _tcmid(s1, g1, dinv, p['conv1_b'], p['bn2_g'], p['bn2_b'],
                p['conv2_W'])
    s2 = _segsum_u(g2, row, col)

    Wea = p['edge_att_W']
    a_att, b_att, hc, ho = _tc4(
        s2, g2, dinv, p['conv2_b'], Wea[:H], Wea[H:], p['edge_att_b'],
        p['node_att_W'], p['node_att_b'],
        p['bnc_g'], p['bnc_b'], p['ctx_W'],
        p['bno_g'], p['bno_b'], p['obj_W'])

    ewc, ewo, deg = _edgeatt_call(a_att.reshape(-1), b_att.reshape(-1),
                                  row, col)
    deg = deg.reshape(NC, 2, N)
    degc_t = jnp.transpose(deg[:, 0])
    dego_t = jnp.transpose(deg[:, 1])

    gc, go, dinvc, dinvo = _tc5(degc_t, dego_t, hc, ho)

    sc = _segsum_w(gc, row, col, ewc)
    so = _segsum_w(go, row, col, ewo)

    xc, xc_logis = _tc6(sc, gc, dinvc, p['ctx_b'],
                        p['c_bn1_g'], p['c_bn1_b'], p['c_fc1_W'], p['c_fc1_b'],
                        p['c_bn2_g'], p['c_bn2_b'], p['c_fc2_W'], p['c_fc2_b'])
    xo, xo_logis = _tc6(so, go, dinvo, p['obj_b'],
                        p['o_bn1_g'], p['o_bn1_b'], p['o_fc1_W'], p['o_fc1_b'],
                        p['o_bn2_g'], p['o_bn2_b'], p['o_fc2_W'], p['o_fc2_b'])

    perm = jax.random.permutation(jax.random.key(42), N).astype(i32)
    xcp = _permgather_call(xc, perm)

    xco_logis = _tc7(xcp, xo,
                     p['co_bn1_g'], p['co_bn1_b'], p['co_fc1_W'],
                     p['co_fc1_b'], p['co_bn2_g'], p['co_bn2_b'],
                     p['co_fc2_W'], p['co_fc2_b'])

    return (xc_logis, xo_logis, xco_logis)
